# Initial kernel scaffold; baseline (speedup 1.0000x reference)
import jax
import jax.numpy as jnp
from jax.experimental import pallas as pl


def kernel(hidden_states, gate_weight, e_score_correction_bias, w_gate, w_up, w_down, ws_gate, ws_up, ws_down):
    raise NotImplementedError("write your pallas kernel here")


# speedup vs baseline: 2.4522x; 2.4522x over previous
import jax
import jax.numpy as jnp
from jax import lax
from jax.experimental import pallas as pl
from jax.experimental.pallas import tpu as pltpu

E = 8
KTOP = 2
NGROUP = 4
GSZ = E // NGROUP
SCALE = 2.5
H = 1024
I = 512
SI = 1024
T = 2048

_TCHUNK = 512


def _silu(v):
    return v / (1.0 + jnp.exp(-v))


def _top1_mask(vals, width):
    m = jnp.max(vals, axis=1, keepdims=True)
    io = lax.broadcasted_iota(jnp.int32, vals.shape, 1)
    idx = jnp.min(jnp.where(vals == m, io, width), axis=1, keepdims=True)
    return io == idx


def _router_body(logits_ref, bias_ref, comb_ref):
    logits = logits_ref[...]
    scores = 1.0 / (1.0 + jnp.exp(-logits))
    swb = scores + bias_ref[...]

    e_i = lax.broadcasted_iota(jnp.int32, (E, NGROUP), 0)
    g_i = lax.broadcasted_iota(jnp.int32, (E, NGROUP), 1)
    m_eg = ((e_i // GSZ) == g_i).astype(jnp.float32)
    g_j = lax.broadcasted_iota(jnp.int32, (NGROUP, E), 0)
    e_j = lax.broadcasted_iota(jnp.int32, (NGROUP, E), 1)
    m_ge = ((e_j // GSZ) == g_j).astype(jnp.float32)

    gs = jnp.dot(swb, m_eg, preferred_element_type=jnp.float32, precision=lax.Precision.HIGHEST)
    p1 = _top1_mask(gs, NGROUP)
    p2 = _top1_mask(jnp.where(p1, -1e30, gs), NGROUP)
    gmask = jnp.logical_or(p1, p2).astype(jnp.float32)
    emask = jnp.dot(gmask, m_ge, preferred_element_type=jnp.float32, precision=lax.Precision.HIGHEST)

    masked = jnp.where(emask > 0.5, swb, -1e9)
    oh1 = _top1_mask(masked, E)
    oh2 = _top1_mask(jnp.where(oh1, -1e30, masked), E)
    w1 = jnp.sum(jnp.where(oh1, scores, 0.0), axis=1, keepdims=True)
    w2 = jnp.sum(jnp.where(oh2, scores, 0.0), axis=1, keepdims=True)
    r = SCALE / (w1 + w2 + 1e-20)
    comb_ref[...] = (jnp.where(oh1, w1, 0.0) + jnp.where(oh2, w2, 0.0)) * r


def _moe_body(comb_ref, x_ref, wg_ref, wu_ref, wd_ref, out_ref):
    e = pl.program_id(0)

    @pl.when(e == 0)
    def _init():
        out_ref[...] = jnp.zeros_like(out_ref)

    wg = wg_ref[0]
    wu = wu_ref[0]
    wd = wd_ref[0]
    lane = lax.broadcasted_iota(jnp.int32, (_TCHUNK, E), 1)
    for c in range(T // _TCHUNK):
        sl = pl.ds(c * _TCHUNK, _TCHUNK)
        x = x_ref[sl, :]
        g = jnp.dot(x, wg, preferred_element_type=jnp.float32)
        u = jnp.dot(x, wu, preferred_element_type=jnp.float32)
        a = _silu(g) * u
        d = jnp.dot(a, wd, preferred_element_type=jnp.float32)
        cb = comb_ref[sl, :]
        col = jnp.sum(jnp.where(lane == e, cb, 0.0), axis=1, keepdims=True)
        out_ref[sl, :] += col * d


def _shared_body(x_ref, wsg_ref, wsu_ref, wsd_ref, routed_ref, out_ref):
    x = x_ref[...]
    g = jnp.dot(x, wsg_ref[...], preferred_element_type=jnp.float32)
    u = jnp.dot(x, wsu_ref[...], preferred_element_type=jnp.float32)
    a = _silu(g) * u
    d = jnp.dot(a, wsd_ref[...], preferred_element_type=jnp.float32)
    out_ref[...] = routed_ref[...] + d


def kernel(hidden_states, gate_weight, e_score_correction_bias,
           w_gate, w_up, w_down, ws_gate, ws_up, ws_down):
    x = hidden_states
    logits = jnp.dot(x, gate_weight.T).astype(jnp.float32)
    bias2 = e_score_correction_bias.reshape(1, E)

    comb = pl.pallas_call(
        _router_body,
        out_shape=jax.ShapeDtypeStruct((T, E), jnp.float32),
        in_specs=[
            pl.BlockSpec((T, E), lambda: (0, 0)),
            pl.BlockSpec((1, E), lambda: (0, 0)),
        ],
        out_specs=pl.BlockSpec((T, E), lambda: (0, 0)),
    )(logits, bias2)

    routed = pl.pallas_call(
        _moe_body,
        grid=(E,),
        out_shape=jax.ShapeDtypeStruct((T, H), jnp.float32),
        in_specs=[
            pl.BlockSpec((T, E), lambda e: (0, 0)),
            pl.BlockSpec((T, H), lambda e: (0, 0)),
            pl.BlockSpec((1, H, I), lambda e: (e, 0, 0)),
            pl.BlockSpec((1, H, I), lambda e: (e, 0, 0)),
            pl.BlockSpec((1, I, H), lambda e: (e, 0, 0)),
        ],
        out_specs=pl.BlockSpec((T, H), lambda e: (0, 0)),
        compiler_params=pltpu.CompilerParams(
            dimension_semantics=("arbitrary",),
        ),
    )(comb, x, w_gate, w_up, w_down)

    tb = 512
    out = pl.pallas_call(
        _shared_body,
        grid=(T // tb,),
        out_shape=jax.ShapeDtypeStruct((T, H), jnp.float32),
        in_specs=[
            pl.BlockSpec((tb, H), lambda i: (i, 0)),
            pl.BlockSpec((H, SI), lambda i: (0, 0)),
            pl.BlockSpec((H, SI), lambda i: (0, 0)),
            pl.BlockSpec((SI, H), lambda i: (0, 0)),
            pl.BlockSpec((tb, H), lambda i: (i, 0)),
        ],
        out_specs=pl.BlockSpec((tb, H), lambda i: (i, 0)),
        compiler_params=pltpu.CompilerParams(
            dimension_semantics=("arbitrary",),
        ),
    )(x, ws_gate, ws_up, ws_down, routed)

    return out
